# baseline (speedup 1.0000x reference)
import functools

import jax
import jax.numpy as jnp
from jax import lax
from jax.experimental import pallas as pl
from jax.experimental.pallas import tpu as pltpu
from jax.experimental.pallas import tpu_sc as plsc

BATCH = 4096
SEQ = 50
EMB_DIM = 64

NC = 2
NS = 16
NW = NC * NS

B_TOTAL = BATCH * SEQ
B_PER_W = B_TOTAL // NW
CHUNK = 128
N_CHUNKS = B_PER_W // CHUNK


def _make_kernel():
  mesh = plsc.VectorSubcoreMesh(
      core_axis_name="c", subcore_axis_name="s",
      num_cores=NC, num_subcores=NS)

  @functools.partial(
      pl.kernel,
      out_type=jax.ShapeDtypeStruct((B_TOTAL, EMB_DIM), jnp.float32),
      mesh=mesh,
      scratch_types=[
          pltpu.VMEM((N_CHUNKS, CHUNK), jnp.int32),
          pltpu.VMEM((CHUNK, EMB_DIM), jnp.float32),
          pltpu.SemaphoreType.DMA,
      ],
      compiler_params=pltpu.CompilerParams(use_tc_tiling_on_sc=False),
  )
  def gather_kernel(idx_hbm, table_hbm, out_hbm, idx_v, rows_v, gsem):
    wid = lax.axis_index("s") * NC + lax.axis_index("c")
    base = wid * B_PER_W
    pltpu.sync_copy(idx_hbm.at[wid], idx_v)

    def chunk_body(j, carry):
      pltpu.async_copy(table_hbm.at[idx_v.at[j]], rows_v, gsem).wait()
      pltpu.sync_copy(rows_v, out_hbm.at[pl.ds(base + j * CHUNK, CHUNK)])
      return carry

    lax.fori_loop(0, N_CHUNKS, chunk_body, 0, unroll=False)

  return gather_kernel


_gather = _make_kernel()


@jax.jit
def kernel(inputs, embeddings):
  idx = inputs.reshape(NW, N_CHUNKS, CHUNK)
  out = _gather(idx, embeddings)
  return out.reshape(BATCH, SEQ, EMB_DIM)


# speedup vs baseline: 1.0439x; 1.0439x over previous
import functools

import jax
import jax.numpy as jnp
from jax import lax
from jax.experimental import pallas as pl
from jax.experimental.pallas import tpu as pltpu
from jax.experimental.pallas import tpu_sc as plsc

BATCH = 4096
SEQ = 50
EMB_DIM = 64

NC = 2
NS = 16
NW = NC * NS

B_TOTAL = BATCH * SEQ
B_PER_W = B_TOTAL // NW
CHUNK = 800
N_CHUNKS = B_PER_W // CHUNK


def _make_kernel():
  mesh = plsc.VectorSubcoreMesh(
      core_axis_name="c", subcore_axis_name="s",
      num_cores=NC, num_subcores=NS)

  @functools.partial(
      pl.kernel,
      out_type=jax.ShapeDtypeStruct((B_TOTAL, EMB_DIM), jnp.float32),
      mesh=mesh,
      scratch_types=[
          pltpu.VMEM((N_CHUNKS, CHUNK), jnp.int32),
          pltpu.VMEM((2, CHUNK, EMB_DIM), jnp.float32),
          pltpu.SemaphoreType.DMA,
          pltpu.SemaphoreType.DMA,
      ],
      compiler_params=pltpu.CompilerParams(use_tc_tiling_on_sc=False),
  )
  def gather_kernel(idx_hbm, table_hbm, out_hbm, idx_v, rows_v, gsem, wsem):
    wid = lax.axis_index("s") * NC + lax.axis_index("c")
    base = wid * B_PER_W
    pltpu.sync_copy(idx_hbm.at[wid], idx_v)

    def fire_gather(j):
      return pltpu.async_copy(
          table_hbm.at[idx_v.at[j]], rows_v.at[j % 2], gsem)

    def fire_write(j):
      return pltpu.async_copy(
          rows_v.at[j % 2], out_hbm.at[pl.ds(base + j * CHUNK, CHUNK)], wsem)

    gd = [None] * N_CHUNKS
    wd = [None] * N_CHUNKS
    gd[0] = fire_gather(0)
    for j in range(N_CHUNKS):
      if j + 1 < N_CHUNKS:
        if j >= 1:
          wd[j - 1].wait()
        gd[j + 1] = fire_gather(j + 1)
      gd[j].wait()
      wd[j] = fire_write(j)
    if N_CHUNKS >= 2:
      wd[N_CHUNKS - 2].wait()
    wd[N_CHUNKS - 1].wait()

  return gather_kernel


_gather = _make_kernel()


@jax.jit
def kernel(inputs, embeddings):
  idx = inputs.reshape(NW, N_CHUNKS, CHUNK)
  out = _gather(idx, embeddings)
  return out.reshape(BATCH, SEQ, EMB_DIM)


# speedup vs baseline: 1.0443x; 1.0004x over previous
import functools

import jax
import jax.numpy as jnp
from jax import lax
from jax.experimental import pallas as pl
from jax.experimental.pallas import tpu as pltpu
from jax.experimental.pallas import tpu_sc as plsc

BATCH = 4096
SEQ = 50
EMB_DIM = 64

NC = 2
NS = 16
NW = NC * NS

B_TOTAL = BATCH * SEQ
B_PER_W = B_TOTAL // NW
CHUNK = 400
N_CHUNKS = B_PER_W // CHUNK
NBUF = 4


def _make_kernel():
  mesh = plsc.VectorSubcoreMesh(
      core_axis_name="c", subcore_axis_name="s",
      num_cores=NC, num_subcores=NS)

  @functools.partial(
      pl.kernel,
      out_type=jax.ShapeDtypeStruct((B_TOTAL, EMB_DIM), jnp.float32),
      mesh=mesh,
      scratch_types=[
          pltpu.VMEM((N_CHUNKS, CHUNK), jnp.int32),
          pltpu.VMEM((NBUF, CHUNK, EMB_DIM), jnp.float32),
          pltpu.SemaphoreType.DMA,
          pltpu.SemaphoreType.DMA,
      ],
      compiler_params=pltpu.CompilerParams(use_tc_tiling_on_sc=False),
  )
  def gather_kernel(idx_hbm, table_hbm, out_hbm, idx_v, rows_v, gsem, wsem):
    wid = lax.axis_index("s") * NC + lax.axis_index("c")
    base = wid * B_PER_W
    pltpu.sync_copy(idx_hbm.at[wid], idx_v)

    def fire_gather(j):
      return pltpu.async_copy(
          table_hbm.at[idx_v.at[j]], rows_v.at[j % NBUF], gsem)

    def fire_write(j):
      return pltpu.async_copy(
          rows_v.at[j % NBUF],
          out_hbm.at[pl.ds(base + j * CHUNK, CHUNK)], wsem)

    DEPTH = NBUF - 1
    gd = [None] * N_CHUNKS
    wd = [None] * N_CHUNKS
    for j in range(min(DEPTH, N_CHUNKS)):
      gd[j] = fire_gather(j)
    for j in range(N_CHUNKS):
      gd[j].wait()
      wd[j] = fire_write(j)
      nxt = j + DEPTH
      if nxt < N_CHUNKS:
        if j >= 1:
          wd[j - 1].wait()
        gd[nxt] = fire_gather(nxt)
    for j in range(max(0, N_CHUNKS - DEPTH - 1), N_CHUNKS):
      if wd[j] is not None:
        wd[j].wait()

  return gather_kernel


_gather = _make_kernel()


@jax.jit
def kernel(inputs, embeddings):
  idx = inputs.reshape(NW, N_CHUNKS, CHUNK)
  out = _gather(idx, embeddings)
  return out.reshape(BATCH, SEQ, EMB_DIM)
